# baseline (speedup 1.0000x reference)
import functools

import jax
import jax.numpy as jnp
from jax import lax
from jax.experimental import pallas as pl
from jax.experimental.pallas import tpu as pltpu
from jax.experimental.pallas import tpu_sc as plsc

N = 10000
E = 320000
D_IN = 128
HID = 192
OUT_CH = 128
OUT_DIM = 10
NUM_GRAPHS = 16
F32 = jnp.float32

NC = 2
NS = 16
NW = NC * NS
CHUNK = 128
NCHUNK = 80
EPW = CHUNK * NCHUNK
EP = NW * EPW
ROWS_PER_TILE = 640
NP = ROWS_PER_TILE * NS
DUMMY = N

_MESH = dict(mesh=plsc.VectorSubcoreMesh(core_axis_name="c", subcore_axis_name="s"))


def _sc_body_deg(src_hbm, ones_hbm, zeros_hbm, out_hbm, sidx, ones_v, acc):
    c = lax.axis_index("c")
    s = lax.axis_index("s")
    w = c * NS + s
    pltpu.sync_copy(src_hbm.at[w], sidx)
    pltpu.sync_copy(ones_hbm, ones_v)
    pltpu.sync_copy(zeros_hbm, acc.at[pl.ds(s * ROWS_PER_TILE, ROWS_PER_TILE)])
    plsc.subcore_barrier()

    def step(j, carry):
        pltpu.sync_copy(ones_v, acc.at[sidx.at[j]], add=True)
        return carry

    lax.fori_loop(0, NCHUNK, step, 0)
    plsc.subcore_barrier()
    pltpu.sync_copy(acc.at[pl.ds(s * ROWS_PER_TILE, ROWS_PER_TILE)],
                    out_hbm.at[c, pl.ds(s * ROWS_PER_TILE, ROWS_PER_TILE)])


def _sc_deg(src3, ones128, zeros128):
    return pl.kernel(
        _sc_body_deg,
        out_type=jax.ShapeDtypeStruct((NC, NP, OUT_CH), F32),
        scratch_types=[
            pltpu.VMEM((NCHUNK, CHUNK), jnp.int32),
            pltpu.VMEM((CHUNK, OUT_CH), F32),
            pltpu.VMEM_SHARED((NP, OUT_CH), F32),
        ],
        **_MESH,
    )(src3, ones128, zeros128)


W = 16
CHUNKS_PER_S = 2 * NCHUNK
C0_CHUNKS = 80
C0_WIN = C0_CHUNKS // W
C1_WIN = (CHUNKS_PER_S - C0_CHUNKS) // W


def _sc_body_prop(a_hbm, src_hbm, dst_hbm, zeros_hbm, out_hbm,
                  sw, dw, bufs, acc, gsems, ssems, rsem):
    c = lax.axis_index("c")
    s = lax.axis_index("s")
    pltpu.sync_copy(zeros_hbm, acc.at[pl.ds(s * ROWS_PER_TILE, ROWS_PER_TILE)])
    plsc.subcore_barrier()

    start = c * C0_CHUNKS
    nwin = C0_WIN - (C0_WIN - C1_WIN) * c

    def fire_g(bank, k, b):
        pltpu.async_copy(a_hbm.at[sw.at[bank, k]], bufs.at[b], gsems.at[b])

    def wait_g(b):
        pltpu.make_async_copy(a_hbm.at[sw.at[0, 0]], bufs.at[b], gsems.at[b]).wait()

    def fire_s(bank, k, b):
        pltpu.async_copy(bufs.at[b], acc.at[dw.at[bank, k]], ssems.at[b], add=True)

    def wait_s(b):
        pltpu.make_async_copy(bufs.at[b], acc.at[dw.at[0, 0]], ssems.at[b]).wait()

    def fire_refill(v, bank):
        off = pl.multiple_of(start + v * W, W)
        pltpu.async_copy(src_hbm.at[s, pl.ds(off, W)], sw.at[bank], rsem)
        pltpu.async_copy(dst_hbm.at[s, pl.ds(off, W)], dw.at[bank], rsem)

    def wait_refill(bank):
        pltpu.make_async_copy(src_hbm.at[s, pl.ds(0, W)], sw.at[bank], rsem).wait()
        pltpu.make_async_copy(dst_hbm.at[s, pl.ds(0, W)], dw.at[bank], rsem).wait()

    def win_steps(v, bank, nbank, first, last):
        for k in range(W):
            b = k % 2
            wait_g(b)
            fire_s(bank, k, b)
            if k == 2:
                @pl.when(jnp.logical_not(last))
                def _():
                    fire_refill(v + 1, nbank)
            if not (first and k == 0):
                wait_s(1 - b)
            if k < W - 1:
                fire_g(bank, k + 1, 1 - b)
            else:
                @pl.when(jnp.logical_not(last))
                def _():
                    wait_refill(nbank)
                    fire_g(nbank, 0, 1 - b)
        return

    pltpu.sync_copy(src_hbm.at[s, pl.ds(pl.multiple_of(start, W), W)], sw.at[0])
    pltpu.sync_copy(dst_hbm.at[s, pl.ds(pl.multiple_of(start, W), W)], dw.at[0])
    fire_g(0, 0, 0)
    win_steps(0, 0, 1, True, jnp.bool_(False))

    def middle(v, carry):
        bank = lax.rem(v, 2)
        win_steps(v, bank, 1 - bank, False, v == nwin - 1)
        return carry

    lax.fori_loop(1, nwin, middle, 0)
    wait_s(1)

    plsc.subcore_barrier()
    pltpu.sync_copy(acc.at[pl.ds(s * ROWS_PER_TILE, ROWS_PER_TILE)],
                    out_hbm.at[c, pl.ds(s * ROWS_PER_TILE, ROWS_PER_TILE)])


def _sc_prop(a, src3, dst3, zeros128):
    return pl.kernel(
        _sc_body_prop,
        out_type=jax.ShapeDtypeStruct((NC, NP, OUT_CH), F32),
        scratch_types=[
            pltpu.VMEM((2, W, CHUNK), jnp.int32),
            pltpu.VMEM((2, W, CHUNK), jnp.int32),
            pltpu.VMEM((2, CHUNK, OUT_CH), F32),
            pltpu.VMEM_SHARED((NP, OUT_CH), F32),
            pltpu.SemaphoreType.DMA((2,)),
            pltpu.SemaphoreType.DMA((2,)),
            pltpu.SemaphoreType.DMA,
        ],
        **_MESH,
    )(a, src3, dst3, zeros128)


BS = 1000
GRID = N // BS


def _s_of(d0_ref, d1_ref):
    deg = d0_ref[:, 0:1] + d1_ref[:, 0:1]
    return jnp.where(deg > 0.0, lax.rsqrt(jnp.maximum(deg, 1e-30)), 0.0)


def _tc_body_xs(x_ref, d0_ref, d1_ref, xs_ref):
    xs_ref[...] = _s_of(d0_ref, d1_ref) * x_ref[...]


def _tc_body_t1s(g0_ref, g1_ref, d0_ref, d1_ref, o_ref):
    s = _s_of(d0_ref, d1_ref)
    o_ref[...] = (-s * s) * (g0_ref[...] + g1_ref[...])


def _dot(a, b):
    return lax.dot_general(a, b, (((1,), (0,)), ((), ())),
                           preferred_element_type=F32,
                           precision=lax.Precision.HIGHEST)


def _dotT(a, b):
    return lax.dot_general(a, b, (((0,), (0,)), ((), ())),
                           preferred_element_type=F32,
                           precision=lax.Precision.HIGHEST)


def _tc_body_layer(x_ref, d0_ref, d1_ref, g1a_ref, g1b_ref, g2a_ref, g2b_ref,
                   oh_ref, w10_ref, w11_ref, w12_ref, b1_ref, w21_ref, w22_ref,
                   a1_ref, a2s_ref, sh1_ref):
    i = pl.program_id(0)
    s = _s_of(d0_ref, d1_ref)
    x = x_ref[...]
    t1 = (-s) * (g1a_ref[...] + g1b_ref[...])
    t2 = (-2.0 * s) * (g2a_ref[...] + g2b_ref[...]) - x
    h1 = (_dot(x, w10_ref[...]) + _dot(t1, w11_ref[...])
          + _dot(t2, w12_ref[...]) + b1_ref[...])
    a1_ref[...] = _dot(h1, w21_ref[...])
    a2s_ref[...] = s * _dot(h1, w22_ref[...])

    @pl.when(i == 0)
    def _():
        sh1_ref[...] = jnp.zeros_like(sh1_ref)

    sh1_ref[...] += _dotT(oh_ref[...], h1)


def _tc_body_es(a1_ref, g0_ref, g1_ref, d0_ref, d1_ref, o_ref):
    s = _s_of(d0_ref, d1_ref)
    o_ref[...] = s * a1_ref[...] - (2.0 * s * s) * (g0_ref[...] + g1_ref[...])


def _tc_body_final(g0_ref, g1_ref, d0_ref, d1_ref, oh_ref, sh1_ref,
                   w2d_ref, b2_ref, wfc_ref, bfc_ref,
                   out_ref, sf_acc, n_acc):
    i = pl.program_id(0)

    @pl.when(i == 0)
    def _():
        sf_acc[...] = jnp.zeros_like(sf_acc)
        n_acc[...] = jnp.zeros_like(n_acc)

    s = _s_of(d0_ref, d1_ref)
    f = (-s) * (g0_ref[...] + g1_ref[...])
    oh = oh_ref[...]
    sf_acc[...] += _dotT(oh, f)
    n_acc[...] += _dotT(oh, jnp.ones_like(f))

    @pl.when(i == GRID - 1)
    def _():
        n = n_acc[...]
        pooled_sum = _dot(sh1_ref[...], w2d_ref[...]) + sf_acc[...] \
            + n * b2_ref[...]
        pooled = pooled_sum / jnp.maximum(n, 1.0)
        logits = _dot(pooled, wfc_ref[...]) + bfc_ref[...]
        m = jnp.max(logits, axis=1, keepdims=True)
        z = logits - m
        lse = jnp.log(jnp.sum(jnp.exp(z), axis=1, keepdims=True))
        out_ref[...] = z - lse


def _row_spec(d):
    return pl.BlockSpec((BS, d), lambda i: (i, 0))


def _full_spec(shape):
    return pl.BlockSpec(shape, lambda i: tuple(0 for _ in shape))


def _tc_call(body, in_specs, out_specs, out_shape, scratch_shapes=()):
    return pl.pallas_call(
        body,
        grid=(GRID,),
        in_specs=in_specs,
        out_specs=out_specs,
        out_shape=out_shape,
        scratch_shapes=list(scratch_shapes),
    )


def kernel(x, edge_index, batch, W1, b1, W2, b2, Wfc, bfc):
    src = edge_index[0]
    dst = edge_index[1]
    pad = EP - E
    pad_cycle = DUMMY + (jnp.arange(pad, dtype=jnp.int32) % (NP - N))
    src_pd = jnp.concatenate([src, pad_cycle])
    src_pg = jnp.concatenate([src, jnp.zeros((pad,), jnp.int32)])
    dst_p = jnp.concatenate([dst, pad_cycle])
    src3 = src_pd.reshape(NW, NCHUNK, CHUNK)
    dst3 = dst_p.reshape(NS, CHUNKS_PER_S, CHUNK)
    srcp3 = src_pg.reshape(NS, CHUNKS_PER_S, CHUNK)
    ones128 = jnp.ones((CHUNK, OUT_CH), F32)
    zeros128 = jnp.zeros((ROWS_PER_TILE, OUT_CH), F32)
    onehot = (batch[:, None] == jnp.arange(NUM_GRAPHS, dtype=jnp.int32)[None, :]).astype(F32)

    degp = _sc_deg(src3, ones128, zeros128)
    d0 = degp[0, :N, :16]
    d1 = degp[1, :N, :16]

    deg_specs = [_row_spec(16), _row_spec(16)]

    xs = _tc_call(_tc_body_xs,
                  [_row_spec(D_IN)] + deg_specs,
                  _row_spec(D_IN),
                  jax.ShapeDtypeStruct((N, D_IN), F32))(x, d0, d1)

    g1p = _sc_prop(xs, srcp3, dst3, zeros128)
    g1a, g1b = g1p[0, :N, :], g1p[1, :N, :]

    t1s = _tc_call(_tc_body_t1s,
                   [_row_spec(D_IN), _row_spec(D_IN)] + deg_specs,
                   _row_spec(D_IN),
                   jax.ShapeDtypeStruct((N, D_IN), F32))(g1a, g1b, d0, d1)

    g2p = _sc_prop(t1s, srcp3, dst3, zeros128)
    g2a, g2b = g2p[0, :N, :], g2p[1, :N, :]

    W10, W11, W12 = W1[0], W1[1], W1[2]
    W21, W22 = W2[1], W2[2]
    W2d = W2[0] - W2[2]

    a1, a2s, sh1 = _tc_call(
        _tc_body_layer,
        [_row_spec(D_IN)] + deg_specs
        + [_row_spec(D_IN)] * 4 + [_row_spec(NUM_GRAPHS)]
        + [_full_spec((D_IN, HID))] * 3 + [_full_spec((1, HID))]
        + [_full_spec((HID, OUT_CH))] * 2,
        [_row_spec(OUT_CH), _row_spec(OUT_CH), _full_spec((NUM_GRAPHS, HID))],
        (jax.ShapeDtypeStruct((N, OUT_CH), F32),
         jax.ShapeDtypeStruct((N, OUT_CH), F32),
         jax.ShapeDtypeStruct((NUM_GRAPHS, HID), F32)),
    )(x, d0, d1, g1a, g1b, g2a, g2b, onehot,
      W10, W11, W12, b1.reshape(1, HID), W21, W22)

    gdp = _sc_prop(a2s, srcp3, dst3, zeros128)
    gda, gdb = gdp[0, :N, :], gdp[1, :N, :]

    es = _tc_call(_tc_body_es,
                  [_row_spec(OUT_CH), _row_spec(OUT_CH), _row_spec(OUT_CH)]
                  + deg_specs,
                  _row_spec(OUT_CH),
                  jax.ShapeDtypeStruct((N, OUT_CH), F32))(a1, gda, gdb, d0, d1)

    gfp = _sc_prop(es, srcp3, dst3, zeros128)
    gfa, gfb = gfp[0, :N, :], gfp[1, :N, :]

    out = pl.pallas_call(
        _tc_body_final,
        grid=(GRID,),
        in_specs=[_row_spec(OUT_CH), _row_spec(OUT_CH)] + deg_specs
        + [_row_spec(NUM_GRAPHS), _full_spec((NUM_GRAPHS, HID)),
           _full_spec((HID, OUT_CH)), _full_spec((1, OUT_CH)),
           _full_spec((OUT_CH, OUT_DIM)), _full_spec((1, OUT_DIM))],
        out_specs=_full_spec((NUM_GRAPHS, OUT_DIM)),
        out_shape=jax.ShapeDtypeStruct((NUM_GRAPHS, OUT_DIM), F32),
        scratch_shapes=[pltpu.VMEM((NUM_GRAPHS, OUT_CH), F32),
                        pltpu.VMEM((NUM_GRAPHS, OUT_CH), F32)],
    )(gfa, gfb, d0, d1, onehot, sh1, W2d,
      b2.reshape(1, OUT_CH), Wfc, bfc.reshape(1, OUT_DIM))

    return out


# speedup vs baseline: 3.0683x; 3.0683x over previous
import functools

import jax
import jax.numpy as jnp
from jax import lax
from jax.experimental import pallas as pl
from jax.experimental.pallas import tpu as pltpu
from jax.experimental.pallas import tpu_sc as plsc

N = 10000
E = 320000
D_IN = 128
HID = 192
OUT_CH = 128
OUT_DIM = 10
NUM_GRAPHS = 16
F32 = jnp.float32

NC = 2
NS = 16
NW = NC * NS
CHUNK = 128
NCHUNK = 80
EPW = CHUNK * NCHUNK
EP = NW * EPW
ROWS_PER_TILE = 640
NP = ROWS_PER_TILE * NS
DUMMY = N

_MESH = dict(mesh=plsc.VectorSubcoreMesh(core_axis_name="c", subcore_axis_name="s"))


def _sc_body_deg(src_hbm, ones_hbm, zeros_hbm, out_hbm, sidx, ones_v, acc):
    c = lax.axis_index("c")
    s = lax.axis_index("s")
    w = c * NS + s
    pltpu.sync_copy(src_hbm.at[w], sidx)
    pltpu.sync_copy(ones_hbm, ones_v)
    pltpu.sync_copy(zeros_hbm, acc.at[pl.ds(s * ROWS_PER_TILE, ROWS_PER_TILE)])
    plsc.subcore_barrier()

    def step(j, carry):
        pltpu.sync_copy(ones_v, acc.at[sidx.at[j]], add=True)
        return carry

    lax.fori_loop(0, NCHUNK, step, 0)
    plsc.subcore_barrier()
    pltpu.sync_copy(acc.at[pl.ds(s * ROWS_PER_TILE, ROWS_PER_TILE)],
                    out_hbm.at[c, pl.ds(s * ROWS_PER_TILE, ROWS_PER_TILE)])


def _sc_deg(src3, ones128, zeros128):
    return pl.kernel(
        _sc_body_deg,
        out_type=jax.ShapeDtypeStruct((NC, NP, OUT_CH), F32),
        scratch_types=[
            pltpu.VMEM((NCHUNK, CHUNK), jnp.int32),
            pltpu.VMEM((CHUNK, OUT_CH), F32),
            pltpu.VMEM_SHARED((NP, OUT_CH), F32),
        ],
        **_MESH,
    )(src3, ones128, zeros128)


W = 16
CHUNKS_PER_S = 2 * NCHUNK
C0_CHUNKS = 80
C0_WIN = C0_CHUNKS // W
C1_WIN = (CHUNKS_PER_S - C0_CHUNKS) // W


def _sc_body_prop(a_hbm, src_hbm, dst_hbm, zeros_hbm, out_hbm,
                  sw, dw, bufs, acc, gsems, ssems, rsem):
    c = lax.axis_index("c")
    s = lax.axis_index("s")
    pltpu.sync_copy(zeros_hbm, acc.at[pl.ds(s * ROWS_PER_TILE, ROWS_PER_TILE)])
    plsc.subcore_barrier()

    start = c * C0_CHUNKS
    nwin = C0_WIN - (C0_WIN - C1_WIN) * c

    def fire_g(bank, k, b):
        pltpu.async_copy(a_hbm.at[sw.at[bank, k]], bufs.at[b], gsems.at[b])

    def wait_g(b):
        pltpu.make_async_copy(a_hbm.at[sw.at[0, 0]], bufs.at[b], gsems.at[b]).wait()

    def fire_s(bank, k, b):
        pltpu.async_copy(bufs.at[b], acc.at[dw.at[bank, k]], ssems.at[b], add=True)

    def wait_s(b):
        pltpu.make_async_copy(bufs.at[b], acc.at[dw.at[0, 0]], ssems.at[b]).wait()

    def fire_refill(v, bank):
        off = pl.multiple_of(start + v * W, W)
        pltpu.async_copy(src_hbm.at[s, pl.ds(off, W)], sw.at[bank], rsem)
        pltpu.async_copy(dst_hbm.at[s, pl.ds(off, W)], dw.at[bank], rsem)

    def wait_refill(bank):
        pltpu.make_async_copy(src_hbm.at[s, pl.ds(0, W)], sw.at[bank], rsem).wait()
        pltpu.make_async_copy(dst_hbm.at[s, pl.ds(0, W)], dw.at[bank], rsem).wait()

    def win_steps(v, bank, nbank, first, last):
        for k in range(W):
            b = k % 2
            wait_g(b)
            fire_s(bank, k, b)
            if k == 2:
                @pl.when(jnp.logical_not(last))
                def _():
                    fire_refill(v + 1, nbank)
            if not (first and k == 0):
                wait_s(1 - b)
            if k < W - 1:
                fire_g(bank, k + 1, 1 - b)
            else:
                @pl.when(jnp.logical_not(last))
                def _():
                    wait_refill(nbank)
                    fire_g(nbank, 0, 1 - b)
        return

    pltpu.sync_copy(src_hbm.at[s, pl.ds(pl.multiple_of(start, W), W)], sw.at[0])
    pltpu.sync_copy(dst_hbm.at[s, pl.ds(pl.multiple_of(start, W), W)], dw.at[0])
    fire_g(0, 0, 0)
    win_steps(0, 0, 1, True, jnp.bool_(False))

    def middle(v, carry):
        bank = lax.rem(v, 2)
        win_steps(v, bank, 1 - bank, False, v == nwin - 1)
        return carry

    lax.fori_loop(1, nwin, middle, 0)
    wait_s(1)

    plsc.subcore_barrier()
    pltpu.sync_copy(acc.at[pl.ds(s * ROWS_PER_TILE, ROWS_PER_TILE)],
                    out_hbm.at[c, pl.ds(s * ROWS_PER_TILE, ROWS_PER_TILE)])


def _sc_prop(a, src3, dst3, zeros128):
    return pl.kernel(
        _sc_body_prop,
        out_type=jax.ShapeDtypeStruct((NC, NP, OUT_CH), F32),
        scratch_types=[
            pltpu.VMEM((2, W, CHUNK), jnp.int32),
            pltpu.VMEM((2, W, CHUNK), jnp.int32),
            pltpu.VMEM((2, CHUNK, OUT_CH), F32),
            pltpu.VMEM_SHARED((NP, OUT_CH), F32),
            pltpu.SemaphoreType.DMA((2,)),
            pltpu.SemaphoreType.DMA((2,)),
            pltpu.SemaphoreType.DMA,
        ],
        **_MESH,
    )(a, src3, dst3, zeros128)


BS = 1000
GRID = N // BS


def _s_of(d0_ref, d1_ref):
    deg = d0_ref[:, 0:1] + d1_ref[:, 0:1]
    return jnp.where(deg > 0.0, lax.rsqrt(jnp.maximum(deg, 1e-30)), 0.0)


def _tc_body_xs(x_ref, d0_ref, d1_ref, xs_ref):
    xs_ref[...] = _s_of(d0_ref, d1_ref) * x_ref[...]


def _tc_body_t1s(g0_ref, g1_ref, d0_ref, d1_ref, o_ref):
    s = _s_of(d0_ref, d1_ref)
    o_ref[...] = (-s * s) * (g0_ref[...] + g1_ref[...])


def _dot(a, b):
    return lax.dot_general(a, b, (((1,), (0,)), ((), ())),
                           preferred_element_type=F32,
                           precision=lax.Precision.HIGHEST)


def _dotT(a, b):
    return lax.dot_general(a, b, (((0,), (0,)), ((), ())),
                           preferred_element_type=F32,
                           precision=lax.Precision.HIGHEST)


def _tc_body_layer(x_ref, d0_ref, d1_ref, g1a_ref, g1b_ref, g2a_ref, g2b_ref,
                   oh_ref, w10_ref, w11_ref, w12_ref, b1_ref, w21_ref, w22_ref,
                   a1_ref, a2s_ref, sh1_ref):
    i = pl.program_id(0)
    s = _s_of(d0_ref, d1_ref)
    x = x_ref[...]
    t1 = (-s) * (g1a_ref[...] + g1b_ref[...])
    t2 = (-2.0 * s) * (g2a_ref[...] + g2b_ref[...]) - x
    h1 = (_dot(x, w10_ref[...]) + _dot(t1, w11_ref[...])
          + _dot(t2, w12_ref[...]) + b1_ref[...])
    a1_ref[...] = _dot(h1, w21_ref[...])
    a2s_ref[...] = s * _dot(h1, w22_ref[...])

    @pl.when(i == 0)
    def _():
        sh1_ref[...] = jnp.zeros_like(sh1_ref)

    sh1_ref[...] += _dotT(oh_ref[...], h1)


def _tc_body_es(a1_ref, g0_ref, g1_ref, d0_ref, d1_ref, o_ref):
    s = _s_of(d0_ref, d1_ref)
    o_ref[...] = s * a1_ref[...] - (2.0 * s * s) * (g0_ref[...] + g1_ref[...])


def _tc_body_final(g0_ref, g1_ref, d0_ref, d1_ref, oh_ref, sh1_ref,
                   w2d_ref, b2_ref, wfc_ref, bfc_ref,
                   out_ref, sf_acc, n_acc):
    i = pl.program_id(0)

    @pl.when(i == 0)
    def _():
        sf_acc[...] = jnp.zeros_like(sf_acc)
        n_acc[...] = jnp.zeros_like(n_acc)

    s = _s_of(d0_ref, d1_ref)
    f = (-s) * (g0_ref[...] + g1_ref[...])
    oh = oh_ref[...]
    sf_acc[...] += _dotT(oh, f)
    n_acc[...] += _dotT(oh, jnp.ones_like(f))

    @pl.when(i == GRID - 1)
    def _():
        n = n_acc[...]
        pooled_sum = _dot(sh1_ref[...], w2d_ref[...]) + sf_acc[...] \
            + n * b2_ref[...]
        pooled = pooled_sum / jnp.maximum(n, 1.0)
        logits = _dot(pooled, wfc_ref[...]) + bfc_ref[...]
        m = jnp.max(logits, axis=1, keepdims=True)
        z = logits - m
        lse = jnp.log(jnp.sum(jnp.exp(z), axis=1, keepdims=True))
        out_ref[...] = z - lse


def _row_spec(d):
    return pl.BlockSpec((BS, d), lambda i: (i, 0))


def _full_spec(shape):
    return pl.BlockSpec(shape, lambda i: tuple(0 for _ in shape))


def _tc_call(body, in_specs, out_specs, out_shape, scratch_shapes=()):
    return pl.pallas_call(
        body,
        grid=(GRID,),
        in_specs=in_specs,
        out_specs=out_specs,
        out_shape=out_shape,
        scratch_shapes=list(scratch_shapes),
    )


def kernel(x, edge_index, batch, W1, b1, W2, b2, Wfc, bfc):
    src = edge_index[0]
    dst = edge_index[1]
    pad = EP - E
    pad_cycle = DUMMY + (jnp.arange(pad, dtype=jnp.int32) % (NP - N))
    src_pd = jnp.concatenate([src, pad_cycle])
    src_pg = jnp.concatenate([src, jnp.arange(pad, dtype=jnp.int32) % N])
    dst_p = jnp.concatenate([dst, pad_cycle])
    src3 = src_pd.reshape(NW, NCHUNK, CHUNK)
    dst3 = dst_p.reshape(NS, CHUNKS_PER_S, CHUNK)
    srcp3 = src_pg.reshape(NS, CHUNKS_PER_S, CHUNK)
    ones128 = jnp.ones((CHUNK, OUT_CH), F32)
    zeros128 = jnp.zeros((ROWS_PER_TILE, OUT_CH), F32)
    onehot = (batch[:, None] == jnp.arange(NUM_GRAPHS, dtype=jnp.int32)[None, :]).astype(F32)

    degp = _sc_deg(src3, ones128, zeros128)
    d0 = degp[0, :N, :16]
    d1 = degp[1, :N, :16]

    deg_specs = [_row_spec(16), _row_spec(16)]

    xs = _tc_call(_tc_body_xs,
                  [_row_spec(D_IN)] + deg_specs,
                  _row_spec(D_IN),
                  jax.ShapeDtypeStruct((N, D_IN), F32))(x, d0, d1)

    g1p = _sc_prop(xs, srcp3, dst3, zeros128)
    g1a, g1b = g1p[0, :N, :], g1p[1, :N, :]

    t1s = _tc_call(_tc_body_t1s,
                   [_row_spec(D_IN), _row_spec(D_IN)] + deg_specs,
                   _row_spec(D_IN),
                   jax.ShapeDtypeStruct((N, D_IN), F32))(g1a, g1b, d0, d1)

    g2p = _sc_prop(t1s, srcp3, dst3, zeros128)
    g2a, g2b = g2p[0, :N, :], g2p[1, :N, :]

    W10, W11, W12 = W1[0], W1[1], W1[2]
    W21, W22 = W2[1], W2[2]
    W2d = W2[0] - W2[2]

    a1, a2s, sh1 = _tc_call(
        _tc_body_layer,
        [_row_spec(D_IN)] + deg_specs
        + [_row_spec(D_IN)] * 4 + [_row_spec(NUM_GRAPHS)]
        + [_full_spec((D_IN, HID))] * 3 + [_full_spec((1, HID))]
        + [_full_spec((HID, OUT_CH))] * 2,
        [_row_spec(OUT_CH), _row_spec(OUT_CH), _full_spec((NUM_GRAPHS, HID))],
        (jax.ShapeDtypeStruct((N, OUT_CH), F32),
         jax.ShapeDtypeStruct((N, OUT_CH), F32),
         jax.ShapeDtypeStruct((NUM_GRAPHS, HID), F32)),
    )(x, d0, d1, g1a, g1b, g2a, g2b, onehot,
      W10, W11, W12, b1.reshape(1, HID), W21, W22)

    gdp = _sc_prop(a2s, srcp3, dst3, zeros128)
    gda, gdb = gdp[0, :N, :], gdp[1, :N, :]

    es = _tc_call(_tc_body_es,
                  [_row_spec(OUT_CH), _row_spec(OUT_CH), _row_spec(OUT_CH)]
                  + deg_specs,
                  _row_spec(OUT_CH),
                  jax.ShapeDtypeStruct((N, OUT_CH), F32))(a1, gda, gdb, d0, d1)

    gfp = _sc_prop(es, srcp3, dst3, zeros128)
    gfa, gfb = gfp[0, :N, :], gfp[1, :N, :]

    out = pl.pallas_call(
        _tc_body_final,
        grid=(GRID,),
        in_specs=[_row_spec(OUT_CH), _row_spec(OUT_CH)] + deg_specs
        + [_row_spec(NUM_GRAPHS), _full_spec((NUM_GRAPHS, HID)),
           _full_spec((HID, OUT_CH)), _full_spec((1, OUT_CH)),
           _full_spec((OUT_CH, OUT_DIM)), _full_spec((1, OUT_DIM))],
        out_specs=_full_spec((NUM_GRAPHS, OUT_DIM)),
        out_shape=jax.ShapeDtypeStruct((NUM_GRAPHS, OUT_DIM), F32),
        scratch_shapes=[pltpu.VMEM((NUM_GRAPHS, OUT_CH), F32),
                        pltpu.VMEM((NUM_GRAPHS, OUT_CH), F32)],
    )(gfa, gfb, d0, d1, onehot, sh1, W2d,
      b2.reshape(1, OUT_CH), Wfc, bfc.reshape(1, OUT_DIM))

    return out


# speedup vs baseline: 3.3615x; 1.0956x over previous
import functools

import jax
import jax.numpy as jnp
from jax import lax
from jax.experimental import pallas as pl
from jax.experimental.pallas import tpu as pltpu
from jax.experimental.pallas import tpu_sc as plsc

N = 10000
E = 320000
D_IN = 128
HID = 192
OUT_CH = 128
OUT_DIM = 10
NUM_GRAPHS = 16
F32 = jnp.float32

NC = 2
NS = 16
NW = NC * NS
CHUNK = 128
NCHUNK = 80
EPW = CHUNK * NCHUNK
EP = NW * EPW
ROWS_PER_TILE = 640
NP = ROWS_PER_TILE * NS
DUMMY = N

_MESH = dict(mesh=plsc.VectorSubcoreMesh(core_axis_name="c", subcore_axis_name="s"))


def _sc_body_deg(src_hbm, ones_hbm, zeros_hbm, out_hbm, sidx, ones_v, acc):
    c = lax.axis_index("c")
    s = lax.axis_index("s")
    w = c * NS + s
    pltpu.sync_copy(src_hbm.at[w], sidx)
    pltpu.sync_copy(ones_hbm, ones_v)
    pltpu.sync_copy(zeros_hbm, acc.at[pl.ds(s * ROWS_PER_TILE, ROWS_PER_TILE)])
    plsc.subcore_barrier()

    def step(j, carry):
        pltpu.sync_copy(ones_v, acc.at[sidx.at[j]], add=True)
        return carry

    lax.fori_loop(0, NCHUNK, step, 0)
    plsc.subcore_barrier()
    pltpu.sync_copy(acc.at[pl.ds(s * ROWS_PER_TILE, ROWS_PER_TILE)],
                    out_hbm.at[c, pl.ds(s * ROWS_PER_TILE, ROWS_PER_TILE)])


def _sc_deg(src3, ones128, zeros128):
    return pl.kernel(
        _sc_body_deg,
        out_type=jax.ShapeDtypeStruct((NC, NP, OUT_CH), F32),
        scratch_types=[
            pltpu.VMEM((NCHUNK, CHUNK), jnp.int32),
            pltpu.VMEM((CHUNK, OUT_CH), F32),
            pltpu.VMEM_SHARED((NP, OUT_CH), F32),
        ],
        **_MESH,
    )(src3, ones128, zeros128)


W = 16
CHUNKS_PER_S = 2 * NCHUNK
C0_CHUNKS = 80
C0_WIN = C0_CHUNKS // W
C1_WIN = (CHUNKS_PER_S - C0_CHUNKS) // W


def _sc_body_prop(a_hbm, src_hbm, dst_hbm, zeros_hbm, out_hbm,
                  sw, dw, bufs, acc, gsems, ssems, rsem):
    c = lax.axis_index("c")
    s = lax.axis_index("s")
    pltpu.sync_copy(zeros_hbm, acc.at[pl.ds(s * ROWS_PER_TILE, ROWS_PER_TILE)])
    plsc.subcore_barrier()

    start = c * C0_CHUNKS
    nwin = C0_WIN - (C0_WIN - C1_WIN) * c

    def fire_g(bank, k, b):
        pltpu.async_copy(a_hbm.at[sw.at[bank, k]], bufs.at[b], gsems.at[b])

    def wait_g(b):
        pltpu.make_async_copy(a_hbm.at[sw.at[0, 0]], bufs.at[b], gsems.at[b]).wait()

    def fire_s(bank, k, b):
        pltpu.async_copy(bufs.at[b], acc.at[dw.at[bank, k]], ssems.at[b], add=True)

    def wait_s(b):
        pltpu.make_async_copy(bufs.at[b], acc.at[dw.at[0, 0]], ssems.at[b]).wait()

    def fire_refill(v, bank):
        off = pl.multiple_of(start + v * W, W)
        pltpu.async_copy(src_hbm.at[s, pl.ds(off, W)], sw.at[bank], rsem)
        pltpu.async_copy(dst_hbm.at[s, pl.ds(off, W)], dw.at[bank], rsem)

    def wait_refill(bank):
        pltpu.make_async_copy(src_hbm.at[s, pl.ds(0, W)], sw.at[bank], rsem).wait()
        pltpu.make_async_copy(dst_hbm.at[s, pl.ds(0, W)], dw.at[bank], rsem).wait()

    def win_steps(v, bank, nbank, first, last):
        for k in range(W):
            b = k % 2
            wait_g(b)
            fire_s(bank, k, b)
            if k == 2:
                @pl.when(jnp.logical_not(last))
                def _():
                    fire_refill(v + 1, nbank)
            if not (first and k == 0):
                wait_s(1 - b)
            if k < W - 1:
                fire_g(bank, k + 1, 1 - b)
            else:
                @pl.when(jnp.logical_not(last))
                def _():
                    wait_refill(nbank)
                    fire_g(nbank, 0, 1 - b)
        return

    pltpu.sync_copy(src_hbm.at[s, pl.ds(pl.multiple_of(start, W), W)], sw.at[0])
    pltpu.sync_copy(dst_hbm.at[s, pl.ds(pl.multiple_of(start, W), W)], dw.at[0])
    fire_g(0, 0, 0)
    win_steps(0, 0, 1, True, jnp.bool_(False))

    def middle(v, carry):
        bank = lax.rem(v, 2)
        win_steps(v, bank, 1 - bank, False, v == nwin - 1)
        return carry

    lax.fori_loop(1, nwin, middle, 0)
    wait_s(1)

    plsc.subcore_barrier()
    pltpu.sync_copy(acc.at[pl.ds(s * ROWS_PER_TILE, ROWS_PER_TILE)],
                    out_hbm.at[c, pl.ds(s * ROWS_PER_TILE, ROWS_PER_TILE)])


def _sc_prop(a, src3, dst3, zeros128):
    return pl.kernel(
        _sc_body_prop,
        out_type=jax.ShapeDtypeStruct((NC, NP, OUT_CH), F32),
        scratch_types=[
            pltpu.VMEM((2, W, CHUNK), jnp.int32),
            pltpu.VMEM((2, W, CHUNK), jnp.int32),
            pltpu.VMEM((2, CHUNK, OUT_CH), F32),
            pltpu.VMEM_SHARED((NP, OUT_CH), F32),
            pltpu.SemaphoreType.DMA((2,)),
            pltpu.SemaphoreType.DMA((2,)),
            pltpu.SemaphoreType.DMA,
        ],
        **_MESH,
    )(a, src3, dst3, zeros128)


BS = 1000
GRID = N // BS


def _s_of(d0_ref, d1_ref):
    deg = d0_ref[:, 0:1] + d1_ref[:, 0:1]
    return jnp.where(deg > 0.0, lax.rsqrt(jnp.maximum(deg, 1e-30)), 0.0)


def _tc_body_xs(x_ref, d0_ref, d1_ref, xs_ref):
    xs_ref[...] = _s_of(d0_ref, d1_ref) * x_ref[...]


def _tc_body_t1s(g0_ref, g1_ref, d0_ref, d1_ref, o_ref):
    s = _s_of(d0_ref, d1_ref)
    o_ref[...] = (-s * s) * (g0_ref[...] + g1_ref[...])


def _dot(a, b):
    return lax.dot_general(a, b, (((1,), (0,)), ((), ())),
                           preferred_element_type=F32,
                           precision=lax.Precision.DEFAULT)


def _dotT(a, b):
    return lax.dot_general(a, b, (((0,), (0,)), ((), ())),
                           preferred_element_type=F32,
                           precision=lax.Precision.DEFAULT)


def _tc_body_layer(x_ref, d0_ref, d1_ref, g1a_ref, g1b_ref, g2a_ref, g2b_ref,
                   oh_ref, w10_ref, w11_ref, w12_ref, b1_ref, w21_ref, w22_ref,
                   a1_ref, a2s_ref, sh1_ref):
    i = pl.program_id(0)
    s = _s_of(d0_ref, d1_ref)
    x = x_ref[...]
    t1 = (-s) * (g1a_ref[...] + g1b_ref[...])
    t2 = (-2.0 * s) * (g2a_ref[...] + g2b_ref[...]) - x
    h1 = (_dot(x, w10_ref[...]) + _dot(t1, w11_ref[...])
          + _dot(t2, w12_ref[...]) + b1_ref[...])
    a1_ref[...] = _dot(h1, w21_ref[...])
    a2s_ref[...] = s * _dot(h1, w22_ref[...])

    @pl.when(i == 0)
    def _():
        sh1_ref[...] = jnp.zeros_like(sh1_ref)

    sh1_ref[...] += _dotT(oh_ref[...], h1)


def _tc_body_es(a1_ref, g0_ref, g1_ref, d0_ref, d1_ref, o_ref):
    s = _s_of(d0_ref, d1_ref)
    o_ref[...] = s * a1_ref[...] - (2.0 * s * s) * (g0_ref[...] + g1_ref[...])


def _tc_body_final(g0_ref, g1_ref, d0_ref, d1_ref, oh_ref, sh1_ref,
                   w2d_ref, b2_ref, wfc_ref, bfc_ref,
                   out_ref, sf_acc, n_acc):
    i = pl.program_id(0)

    @pl.when(i == 0)
    def _():
        sf_acc[...] = jnp.zeros_like(sf_acc)
        n_acc[...] = jnp.zeros_like(n_acc)

    s = _s_of(d0_ref, d1_ref)
    f = (-s) * (g0_ref[...] + g1_ref[...])
    oh = oh_ref[...]
    sf_acc[...] += _dotT(oh, f)
    n_acc[...] += _dotT(oh, jnp.ones_like(f))

    @pl.when(i == GRID - 1)
    def _():
        n = n_acc[...]
        pooled_sum = _dot(sh1_ref[...], w2d_ref[...]) + sf_acc[...] \
            + n * b2_ref[...]
        pooled = pooled_sum / jnp.maximum(n, 1.0)
        logits = _dot(pooled, wfc_ref[...]) + bfc_ref[...]
        m = jnp.max(logits, axis=1, keepdims=True)
        z = logits - m
        lse = jnp.log(jnp.sum(jnp.exp(z), axis=1, keepdims=True))
        out_ref[...] = z - lse


def _row_spec(d):
    return pl.BlockSpec((BS, d), lambda i: (i, 0))


def _full_spec(shape):
    return pl.BlockSpec(shape, lambda i: tuple(0 for _ in shape))


def _tc_call(body, in_specs, out_specs, out_shape, scratch_shapes=()):
    return pl.pallas_call(
        body,
        grid=(GRID,),
        in_specs=in_specs,
        out_specs=out_specs,
        out_shape=out_shape,
        scratch_shapes=list(scratch_shapes),
    )


def kernel(x, edge_index, batch, W1, b1, W2, b2, Wfc, bfc):
    src = edge_index[0]
    dst = edge_index[1]
    pad = EP - E
    pad_cycle = DUMMY + (jnp.arange(pad, dtype=jnp.int32) % (NP - N))
    src_pd = jnp.concatenate([src, pad_cycle])
    src_pg = jnp.concatenate([src, jnp.arange(pad, dtype=jnp.int32) % N])
    dst_p = jnp.concatenate([dst, pad_cycle])
    src3 = src_pd.reshape(NW, NCHUNK, CHUNK)
    dst3 = dst_p.reshape(NS, CHUNKS_PER_S, CHUNK)
    srcp3 = src_pg.reshape(NS, CHUNKS_PER_S, CHUNK)
    ones128 = jnp.ones((CHUNK, OUT_CH), F32)
    zeros128 = jnp.zeros((ROWS_PER_TILE, OUT_CH), F32)
    onehot = (batch[:, None] == jnp.arange(NUM_GRAPHS, dtype=jnp.int32)[None, :]).astype(F32)

    degp = _sc_deg(src3, ones128, zeros128)
    d0 = degp[0, :N, :16]
    d1 = degp[1, :N, :16]

    deg_specs = [_row_spec(16), _row_spec(16)]

    xs = _tc_call(_tc_body_xs,
                  [_row_spec(D_IN)] + deg_specs,
                  _row_spec(D_IN),
                  jax.ShapeDtypeStruct((N, D_IN), F32))(x, d0, d1)

    g1p = _sc_prop(xs, srcp3, dst3, zeros128)
    g1a, g1b = g1p[0, :N, :], g1p[1, :N, :]

    t1s = _tc_call(_tc_body_t1s,
                   [_row_spec(D_IN), _row_spec(D_IN)] + deg_specs,
                   _row_spec(D_IN),
                   jax.ShapeDtypeStruct((N, D_IN), F32))(g1a, g1b, d0, d1)

    g2p = _sc_prop(t1s, srcp3, dst3, zeros128)
    g2a, g2b = g2p[0, :N, :], g2p[1, :N, :]

    W10, W11, W12 = W1[0], W1[1], W1[2]
    W21, W22 = W2[1], W2[2]
    W2d = W2[0] - W2[2]

    a1, a2s, sh1 = _tc_call(
        _tc_body_layer,
        [_row_spec(D_IN)] + deg_specs
        + [_row_spec(D_IN)] * 4 + [_row_spec(NUM_GRAPHS)]
        + [_full_spec((D_IN, HID))] * 3 + [_full_spec((1, HID))]
        + [_full_spec((HID, OUT_CH))] * 2,
        [_row_spec(OUT_CH), _row_spec(OUT_CH), _full_spec((NUM_GRAPHS, HID))],
        (jax.ShapeDtypeStruct((N, OUT_CH), F32),
         jax.ShapeDtypeStruct((N, OUT_CH), F32),
         jax.ShapeDtypeStruct((NUM_GRAPHS, HID), F32)),
    )(x, d0, d1, g1a, g1b, g2a, g2b, onehot,
      W10, W11, W12, b1.reshape(1, HID), W21, W22)

    gdp = _sc_prop(a2s, srcp3, dst3, zeros128)
    gda, gdb = gdp[0, :N, :], gdp[1, :N, :]

    es = _tc_call(_tc_body_es,
                  [_row_spec(OUT_CH), _row_spec(OUT_CH), _row_spec(OUT_CH)]
                  + deg_specs,
                  _row_spec(OUT_CH),
                  jax.ShapeDtypeStruct((N, OUT_CH), F32))(a1, gda, gdb, d0, d1)

    gfp = _sc_prop(es, srcp3, dst3, zeros128)
    gfa, gfb = gfp[0, :N, :], gfp[1, :N, :]

    out = pl.pallas_call(
        _tc_body_final,
        grid=(GRID,),
        in_specs=[_row_spec(OUT_CH), _row_spec(OUT_CH)] + deg_specs
        + [_row_spec(NUM_GRAPHS), _full_spec((NUM_GRAPHS, HID)),
           _full_spec((HID, OUT_CH)), _full_spec((1, OUT_CH)),
           _full_spec((OUT_CH, OUT_DIM)), _full_spec((1, OUT_DIM))],
        out_specs=_full_spec((NUM_GRAPHS, OUT_DIM)),
        out_shape=jax.ShapeDtypeStruct((NUM_GRAPHS, OUT_DIM), F32),
        scratch_shapes=[pltpu.VMEM((NUM_GRAPHS, OUT_CH), F32),
                        pltpu.VMEM((NUM_GRAPHS, OUT_CH), F32)],
    )(gfa, gfb, d0, d1, onehot, sh1, W2d,
      b2.reshape(1, OUT_CH), Wfc, bfc.reshape(1, OUT_DIM))

    return out


# speedup vs baseline: 3.3668x; 1.0016x over previous
import functools

import jax
import jax.numpy as jnp
from jax import lax
from jax.experimental import pallas as pl
from jax.experimental.pallas import tpu as pltpu
from jax.experimental.pallas import tpu_sc as plsc

N = 10000
E = 320000
D_IN = 128
HID = 192
OUT_CH = 128
OUT_DIM = 10
NUM_GRAPHS = 16
F32 = jnp.float32

NC = 2
NS = 16
NW = NC * NS
CHUNK = 128
NCHUNK = 80
EPW = CHUNK * NCHUNK
EP = NW * EPW
ROWS_PER_TILE = 640
NP = ROWS_PER_TILE * NS
DUMMY = N

_MESH = dict(mesh=plsc.VectorSubcoreMesh(core_axis_name="c", subcore_axis_name="s"))


def _sc_body_deg(src_hbm, ones_hbm, zeros_hbm, out_hbm, sidx, ones_v, acc):
    c = lax.axis_index("c")
    s = lax.axis_index("s")
    w = c * NS + s
    pltpu.sync_copy(src_hbm.at[w], sidx)
    pltpu.sync_copy(ones_hbm, ones_v)
    pltpu.sync_copy(zeros_hbm, acc.at[pl.ds(s * ROWS_PER_TILE, ROWS_PER_TILE)])
    plsc.subcore_barrier()

    def step(j, carry):
        pltpu.sync_copy(ones_v, acc.at[sidx.at[j]], add=True)
        return carry

    lax.fori_loop(0, NCHUNK, step, 0)
    plsc.subcore_barrier()
    pltpu.sync_copy(acc.at[pl.ds(s * ROWS_PER_TILE, ROWS_PER_TILE)],
                    out_hbm.at[c, pl.ds(s * ROWS_PER_TILE, ROWS_PER_TILE)])


DEG_W = 128


def _sc_deg(src3, ones_w, zeros_w):
    return pl.kernel(
        _sc_body_deg,
        out_type=jax.ShapeDtypeStruct((NC, NP, DEG_W), F32),
        scratch_types=[
            pltpu.VMEM((NCHUNK, CHUNK), jnp.int32),
            pltpu.VMEM((CHUNK, DEG_W), F32),
            pltpu.VMEM_SHARED((NP, DEG_W), F32),
        ],
        **_MESH,
    )(src3, ones_w, zeros_w)


W = 16
CHUNKS_PER_S = 2 * NCHUNK
C0_CHUNKS = 80
C0_WIN = C0_CHUNKS // W
C1_WIN = (CHUNKS_PER_S - C0_CHUNKS) // W


def _sc_body_prop(a_hbm, src_hbm, dst_hbm, zeros_hbm, out_hbm,
                  sw, dw, bufs, acc, gsems, ssems, rsem):
    c = lax.axis_index("c")
    s = lax.axis_index("s")
    pltpu.sync_copy(zeros_hbm, acc.at[pl.ds(s * ROWS_PER_TILE, ROWS_PER_TILE)])
    plsc.subcore_barrier()

    start = c * C0_CHUNKS
    nwin = C0_WIN - (C0_WIN - C1_WIN) * c

    def fire_g(bank, k, b):
        pltpu.async_copy(a_hbm.at[sw.at[bank, k]], bufs.at[b], gsems.at[b])

    def wait_g(b):
        pltpu.make_async_copy(a_hbm.at[sw.at[0, 0]], bufs.at[b], gsems.at[b]).wait()

    def fire_s(bank, k, b):
        pltpu.async_copy(bufs.at[b], acc.at[dw.at[bank, k]], ssems.at[b], add=True)

    def wait_s(b):
        pltpu.make_async_copy(bufs.at[b], acc.at[dw.at[0, 0]], ssems.at[b]).wait()

    def fire_refill(v, bank):
        off = pl.multiple_of(start + v * W, W)
        pltpu.async_copy(src_hbm.at[s, pl.ds(off, W)], sw.at[bank], rsem)
        pltpu.async_copy(dst_hbm.at[s, pl.ds(off, W)], dw.at[bank], rsem)

    def wait_refill(bank):
        pltpu.make_async_copy(src_hbm.at[s, pl.ds(0, W)], sw.at[bank], rsem).wait()
        pltpu.make_async_copy(dst_hbm.at[s, pl.ds(0, W)], dw.at[bank], rsem).wait()

    def win_steps(v, bank, nbank, first, last):
        for k in range(W):
            b = k % 2
            wait_g(b)
            fire_s(bank, k, b)
            if k == 2:
                @pl.when(jnp.logical_not(last))
                def _():
                    fire_refill(v + 1, nbank)
            if not (first and k == 0):
                wait_s(1 - b)
            if k < W - 1:
                fire_g(bank, k + 1, 1 - b)
            else:
                @pl.when(jnp.logical_not(last))
                def _():
                    wait_refill(nbank)
                    fire_g(nbank, 0, 1 - b)
        return

    pltpu.sync_copy(src_hbm.at[s, pl.ds(pl.multiple_of(start, W), W)], sw.at[0])
    pltpu.sync_copy(dst_hbm.at[s, pl.ds(pl.multiple_of(start, W), W)], dw.at[0])
    fire_g(0, 0, 0)
    win_steps(0, 0, 1, True, jnp.bool_(False))

    def middle(v, carry):
        bank = lax.rem(v, 2)
        win_steps(v, bank, 1 - bank, False, v == nwin - 1)
        return carry

    lax.fori_loop(1, nwin, middle, 0)
    wait_s(1)

    plsc.subcore_barrier()
    pltpu.sync_copy(acc.at[pl.ds(s * ROWS_PER_TILE, ROWS_PER_TILE)],
                    out_hbm.at[c, pl.ds(s * ROWS_PER_TILE, ROWS_PER_TILE)])


def _sc_prop(a, src3, dst3, zeros128):
    return pl.kernel(
        _sc_body_prop,
        out_type=jax.ShapeDtypeStruct((NC, NP, OUT_CH), F32),
        scratch_types=[
            pltpu.VMEM((2, W, CHUNK), jnp.int32),
            pltpu.VMEM((2, W, CHUNK), jnp.int32),
            pltpu.VMEM((2, CHUNK, OUT_CH), F32),
            pltpu.VMEM_SHARED((NP, OUT_CH), F32),
            pltpu.SemaphoreType.DMA((2,)),
            pltpu.SemaphoreType.DMA((2,)),
            pltpu.SemaphoreType.DMA,
        ],
        **_MESH,
    )(a, src3, dst3, zeros128)


BS = 1000
GRID = N // BS


def _s_of(d0_ref, d1_ref):
    deg = d0_ref[:, 0:1] + d1_ref[:, 0:1]
    return jnp.where(deg > 0.0, lax.rsqrt(jnp.maximum(deg, 1e-30)), 0.0)


def _tc_body_xs(x_ref, d0_ref, d1_ref, xs_ref):
    xs_ref[...] = _s_of(d0_ref, d1_ref) * x_ref[...]


def _tc_body_t1s(g0_ref, g1_ref, d0_ref, d1_ref, o_ref):
    s = _s_of(d0_ref, d1_ref)
    o_ref[...] = (-s * s) * (g0_ref[...] + g1_ref[...])


def _dot(a, b):
    return lax.dot_general(a, b, (((1,), (0,)), ((), ())),
                           preferred_element_type=F32,
                           precision=lax.Precision.DEFAULT)


def _dotT(a, b):
    return lax.dot_general(a, b, (((0,), (0,)), ((), ())),
                           preferred_element_type=F32,
                           precision=lax.Precision.DEFAULT)


def _tc_body_layer1(x_ref, d0_ref, d1_ref, g1a_ref, g1b_ref, g2a_ref, g2b_ref,
                    w10_ref, w11_ref, w12_ref, b1_ref, w22_ref,
                    h1_ref, a2s_ref):
    s = _s_of(d0_ref, d1_ref)
    x = x_ref[...]
    t1 = (-s) * (g1a_ref[...] + g1b_ref[...])
    t2 = (-2.0 * s) * (g2a_ref[...] + g2b_ref[...]) - x
    h1 = (_dot(x, w10_ref[...]) + _dot(t1, w11_ref[...])
          + _dot(t2, w12_ref[...]) + b1_ref[...])
    h1_ref[...] = h1
    a2s_ref[...] = s * _dot(h1, w22_ref[...])


def _tc_body_layer2(h1_ref, oh_ref, w21_ref, a1_ref, sh1_ref):
    i = pl.program_id(0)
    h1 = h1_ref[...]
    a1_ref[...] = _dot(h1, w21_ref[...])

    @pl.when(i == 0)
    def _():
        sh1_ref[...] = jnp.zeros_like(sh1_ref)

    sh1_ref[...] += _dotT(oh_ref[...], h1)


def _tc_body_es(a1_ref, g0_ref, g1_ref, d0_ref, d1_ref, o_ref):
    s = _s_of(d0_ref, d1_ref)
    o_ref[...] = s * a1_ref[...] - (2.0 * s * s) * (g0_ref[...] + g1_ref[...])


def _tc_body_final(g0_ref, g1_ref, d0_ref, d1_ref, oh_ref, sh1_ref,
                   w2d_ref, b2_ref, wfc_ref, bfc_ref,
                   out_ref, sf_acc, n_acc):
    i = pl.program_id(0)

    @pl.when(i == 0)
    def _():
        sf_acc[...] = jnp.zeros_like(sf_acc)
        n_acc[...] = jnp.zeros_like(n_acc)

    s = _s_of(d0_ref, d1_ref)
    f = (-s) * (g0_ref[...] + g1_ref[...])
    oh = oh_ref[...]
    sf_acc[...] += _dotT(oh, f)
    n_acc[...] += _dotT(oh, jnp.ones_like(f))

    @pl.when(i == GRID - 1)
    def _():
        n = n_acc[...]
        pooled_sum = _dot(sh1_ref[...], w2d_ref[...]) + sf_acc[...] \
            + n * b2_ref[...]
        pooled = pooled_sum / jnp.maximum(n, 1.0)
        logits = _dot(pooled, wfc_ref[...]) + bfc_ref[...]
        m = jnp.max(logits, axis=1, keepdims=True)
        z = logits - m
        lse = jnp.log(jnp.sum(jnp.exp(z), axis=1, keepdims=True))
        out_ref[...] = z - lse


def _row_spec(d):
    return pl.BlockSpec((BS, d), lambda i: (i, 0))


def _full_spec(shape):
    return pl.BlockSpec(shape, lambda i: tuple(0 for _ in shape))


def _tc_call(body, in_specs, out_specs, out_shape, scratch_shapes=()):
    return pl.pallas_call(
        body,
        grid=(GRID,),
        in_specs=in_specs,
        out_specs=out_specs,
        out_shape=out_shape,
        scratch_shapes=list(scratch_shapes),
    )


def kernel(x, edge_index, batch, W1, b1, W2, b2, Wfc, bfc):
    src = edge_index[0]
    dst = edge_index[1]
    pad = EP - E
    pad_cycle = DUMMY + (jnp.arange(pad, dtype=jnp.int32) % (NP - N))
    src_pd = jnp.concatenate([src, pad_cycle])
    src_pg = jnp.concatenate([src, jnp.arange(pad, dtype=jnp.int32) % N])
    dst_p = jnp.concatenate([dst, pad_cycle])
    src3 = src_pd.reshape(NW, NCHUNK, CHUNK)
    dst3 = dst_p.reshape(NS, CHUNKS_PER_S, CHUNK)
    srcp3 = src_pg.reshape(NS, CHUNKS_PER_S, CHUNK)
    ones_w = jnp.ones((CHUNK, DEG_W), F32)
    zeros_w = jnp.zeros((ROWS_PER_TILE, DEG_W), F32)
    zeros128 = jnp.zeros((ROWS_PER_TILE, OUT_CH), F32)
    onehot = (batch[:, None] == jnp.arange(NUM_GRAPHS, dtype=jnp.int32)[None, :]).astype(F32)

    degp = _sc_deg(src3, ones_w, zeros_w)
    d0 = degp[0, :N, :16]
    d1 = degp[1, :N, :16]

    deg_specs = [_row_spec(16), _row_spec(16)]

    xs = _tc_call(_tc_body_xs,
                  [_row_spec(D_IN)] + deg_specs,
                  _row_spec(D_IN),
                  jax.ShapeDtypeStruct((N, D_IN), F32))(x, d0, d1)

    g1p = _sc_prop(xs, srcp3, dst3, zeros128)
    g1a, g1b = g1p[0, :N, :], g1p[1, :N, :]

    t1s = _tc_call(_tc_body_t1s,
                   [_row_spec(D_IN), _row_spec(D_IN)] + deg_specs,
                   _row_spec(D_IN),
                   jax.ShapeDtypeStruct((N, D_IN), F32))(g1a, g1b, d0, d1)

    g2p = _sc_prop(t1s, srcp3, dst3, zeros128)
    g2a, g2b = g2p[0, :N, :], g2p[1, :N, :]

    W10, W11, W12 = W1[0], W1[1], W1[2]
    W21, W22 = W2[1], W2[2]
    W2d = W2[0] - W2[2]

    h1, a2s = _tc_call(
        _tc_body_layer1,
        [_row_spec(D_IN)] + deg_specs + [_row_spec(D_IN)] * 4
        + [_full_spec((D_IN, HID))] * 3 + [_full_spec((1, HID))]
        + [_full_spec((HID, OUT_CH))],
        [_row_spec(HID), _row_spec(OUT_CH)],
        (jax.ShapeDtypeStruct((N, HID), F32),
         jax.ShapeDtypeStruct((N, OUT_CH), F32)),
    )(x, d0, d1, g1a, g1b, g2a, g2b,
      W10, W11, W12, b1.reshape(1, HID), W22)

    gdp = _sc_prop(a2s, srcp3, dst3, zeros128)
    gda, gdb = gdp[0, :N, :], gdp[1, :N, :]

    a1, sh1 = _tc_call(
        _tc_body_layer2,
        [_row_spec(HID), _row_spec(NUM_GRAPHS), _full_spec((HID, OUT_CH))],
        [_row_spec(OUT_CH), _full_spec((NUM_GRAPHS, HID))],
        (jax.ShapeDtypeStruct((N, OUT_CH), F32),
         jax.ShapeDtypeStruct((NUM_GRAPHS, HID), F32)),
    )(h1, onehot, W21)

    es = _tc_call(_tc_body_es,
                  [_row_spec(OUT_CH), _row_spec(OUT_CH), _row_spec(OUT_CH)]
                  + deg_specs,
                  _row_spec(OUT_CH),
                  jax.ShapeDtypeStruct((N, OUT_CH), F32))(a1, gda, gdb, d0, d1)

    gfp = _sc_prop(es, srcp3, dst3, zeros128)
    gfa, gfb = gfp[0, :N, :], gfp[1, :N, :]

    out = pl.pallas_call(
        _tc_body_final,
        grid=(GRID,),
        in_specs=[_row_spec(OUT_CH), _row_spec(OUT_CH)] + deg_specs
        + [_row_spec(NUM_GRAPHS), _full_spec((NUM_GRAPHS, HID)),
           _full_spec((HID, OUT_CH)), _full_spec((1, OUT_CH)),
           _full_spec((OUT_CH, OUT_DIM)), _full_spec((1, OUT_DIM))],
        out_specs=_full_spec((NUM_GRAPHS, OUT_DIM)),
        out_shape=jax.ShapeDtypeStruct((NUM_GRAPHS, OUT_DIM), F32),
        scratch_shapes=[pltpu.VMEM((NUM_GRAPHS, OUT_CH), F32),
                        pltpu.VMEM((NUM_GRAPHS, OUT_CH), F32)],
    )(gfa, gfb, d0, d1, onehot, sh1, W2d,
      b2.reshape(1, OUT_CH), Wfc, bfc.reshape(1, OUT_DIM))

    return out
